# baseline (speedup 1.0000x reference)
import functools

import jax
import jax.numpy as jnp
from jax import lax
from jax.experimental import pallas as pl
from jax.experimental.pallas import tpu as pltpu
from jax.experimental.pallas import tpu_sc as plsc

B = 16
NV = 6890
H = 512
W = 512
C = 3
P = H * W

_info = plsc.get_sparse_core_info()
NC = _info.num_cores
NS = _info.num_subcores
L = _info.num_lanes
NW = NC * NS
PPW = P // NW
NVP = ((NV * C + 15) // 16) * 16


def _sc_body(verts_hbm, idx_hbm, bary_hbm, out_hbm, idx_v, bary_v, vbuf, obuf):
    wid = lax.axis_index("s") * NC + lax.axis_index("c")
    base_px = wid * PPW

    iota = lax.iota(jnp.int32, L)
    iota3 = iota * 3

    pltpu.sync_copy(idx_hbm.at[pl.ds(base_px * C, PPW * C)], obuf)

    def deint_idx(i, _):
        s = i * L
        for k in range(C):
            g = plsc.load_gather(obuf, [iota3 + (i * (L * C) + k)])
            idx_v[k, pl.ds(s, L)] = g
        return _

    lax.fori_loop(0, PPW // L, deint_idx, 0, unroll=4)

    pltpu.sync_copy(bary_hbm.at[pl.ds(base_px * C, PPW * C)], obuf)

    def deint_bary(i, _):
        s = i * L
        for k in range(C):
            g = plsc.load_gather(obuf, [iota3 + (i * (L * C) + k)])
            bary_v[k, pl.ds(s, L)] = plsc.bitcast(g, jnp.float32)
        return _

    lax.fori_loop(0, PPW // L, deint_bary, 0, unroll=4)

    def px_body(i, _):
        s = i * L
        i0 = idx_v[0, pl.ds(s, L)]
        i1 = idx_v[1, pl.ds(s, L)]
        i2 = idx_v[2, pl.ds(s, L)]
        b0 = bary_v[0, pl.ds(s, L)]
        b1 = bary_v[1, pl.ds(s, L)]
        b2 = bary_v[2, pl.ds(s, L)]
        for c in range(C):
            g0 = plsc.load_gather(vbuf, [i0 + c])
            g1 = plsc.load_gather(vbuf, [i1 + c])
            g2 = plsc.load_gather(vbuf, [i2 + c])
            acc = b0 * g0 + b1 * g1 + b2 * g2
            plsc.store_scatter(
                obuf, [iota3 + (i * (L * C) + c)], plsc.bitcast(acc, jnp.int32)
            )
        return _

    for b in range(B):
        pltpu.sync_copy(verts_hbm.at[b], vbuf)
        lax.fori_loop(0, PPW // L, px_body, 0, unroll=2)
        pltpu.sync_copy(obuf, out_hbm.at[b, pl.ds(base_px * C, PPW * C)])


@functools.partial(jax.jit, static_argnames=())
def kernel(verts, bary_weights, v_index):
    idx3 = v_index.reshape(P * C).astype(jnp.int32) * 3
    bary_bits = bary_weights.reshape(P * C).view(jnp.int32)
    verts_flat = jnp.pad(
        verts.reshape(B, NV * C), ((0, 0), (0, NVP - NV * C))
    )

    sc = pl.kernel(
        _sc_body,
        mesh=plsc.VectorSubcoreMesh(core_axis_name="c", subcore_axis_name="s"),
        out_type=jax.ShapeDtypeStruct((B, P * C), jnp.int32),
        scratch_types=[
            pltpu.VMEM((C, PPW), jnp.int32),
            pltpu.VMEM((C, PPW), jnp.float32),
            pltpu.VMEM((NVP,), jnp.float32),
            pltpu.VMEM((PPW * C,), jnp.int32),
        ],
        compiler_params=pltpu.CompilerParams(needs_layout_passes=False),
    )
    out = sc(verts_flat, idx3, bary_bits)
    return out.view(jnp.float32).reshape(B, H, W, C)


# speedup vs baseline: 1.0039x; 1.0039x over previous
import functools

import jax
import jax.numpy as jnp
from jax import lax
from jax.experimental import pallas as pl
from jax.experimental.pallas import tpu as pltpu
from jax.experimental.pallas import tpu_sc as plsc

B = 16
NV = 6890
H = 512
W = 512
C = 3
P = H * W

_info = plsc.get_sparse_core_info()
NC = _info.num_cores
NS = _info.num_subcores
L = _info.num_lanes
NW = NC * NS
PPW = P // NW
NVP = ((NV * C + 15) // 16) * 16


def _sc_body(verts_hbm, idx_hbm, bary_hbm, out_hbm, idx_v, bary_v, vbuf, obuf):
    wid = lax.axis_index("s") * NC + lax.axis_index("c")
    base_px = wid * PPW

    iota = lax.iota(jnp.int32, L)
    iota3 = iota * 3

    pltpu.sync_copy(idx_hbm.at[pl.ds(base_px * C, PPW * C)], obuf)

    def deint_idx(i, _):
        s = i * L
        for k in range(C):
            g = plsc.load_gather(obuf, [iota3 + (i * (L * C) + k)])
            idx_v[k, pl.ds(s, L)] = g * 3
        return _

    lax.fori_loop(0, PPW // L, deint_idx, 0, unroll=4)

    pltpu.sync_copy(bary_hbm.at[pl.ds(base_px * C, PPW * C)], obuf)

    def deint_bary(i, _):
        s = i * L
        for k in range(C):
            g = plsc.load_gather(obuf, [iota3 + (i * (L * C) + k)])
            bary_v[k, pl.ds(s, L)] = plsc.bitcast(g, jnp.float32)
        return _

    lax.fori_loop(0, PPW // L, deint_bary, 0, unroll=4)

    def px_body(i, _):
        s = i * L
        i0 = idx_v[0, pl.ds(s, L)]
        i1 = idx_v[1, pl.ds(s, L)]
        i2 = idx_v[2, pl.ds(s, L)]
        b0 = bary_v[0, pl.ds(s, L)]
        b1 = bary_v[1, pl.ds(s, L)]
        b2 = bary_v[2, pl.ds(s, L)]
        for c in range(C):
            g0 = plsc.load_gather(vbuf, [i0 + c])
            g1 = plsc.load_gather(vbuf, [i1 + c])
            g2 = plsc.load_gather(vbuf, [i2 + c])
            acc = b0 * g0 + b1 * g1 + b2 * g2
            plsc.store_scatter(
                obuf, [iota3 + (i * (L * C) + c)], plsc.bitcast(acc, jnp.int32)
            )
        return _

    for b in range(B):
        pltpu.sync_copy(verts_hbm.at[b], vbuf)
        lax.fori_loop(0, PPW // L, px_body, 0, unroll=2)
        pltpu.sync_copy(obuf, out_hbm.at[b, pl.ds(base_px * C, PPW * C)])


@functools.partial(jax.jit, static_argnames=())
def kernel(verts, bary_weights, v_index):
    idx_flat = v_index.reshape(P * C).astype(jnp.int32)
    bary_bits = bary_weights.reshape(P * C).view(jnp.int32)
    verts_flat = verts.reshape(B, NV * C)

    sc = pl.kernel(
        _sc_body,
        mesh=plsc.VectorSubcoreMesh(core_axis_name="c", subcore_axis_name="s"),
        out_type=jax.ShapeDtypeStruct((B, P * C), jnp.int32),
        scratch_types=[
            pltpu.VMEM((C, PPW), jnp.int32),
            pltpu.VMEM((C, PPW), jnp.float32),
            pltpu.VMEM((NV * C,), jnp.float32),
            pltpu.VMEM((PPW * C,), jnp.int32),
        ],
        compiler_params=pltpu.CompilerParams(needs_layout_passes=False),
    )
    out = sc(verts_flat, idx_flat, bary_bits)
    return out.view(jnp.float32).reshape(B, H, W, C)


# speedup vs baseline: 1.1692x; 1.1646x over previous
import functools

import jax
import jax.numpy as jnp
from jax import lax
from jax.experimental import pallas as pl
from jax.experimental.pallas import tpu as pltpu
from jax.experimental.pallas import tpu_sc as plsc

B = 16
NV = 6890
H = 512
W = 512
C = 3
P = H * W

_info = plsc.get_sparse_core_info()
NC = _info.num_cores
NS = _info.num_subcores
L = _info.num_lanes
NW = NC * NS
PPW = P // NW
HPW = H // NW
NVP = ((NV * C + 15) // 16) * 16
NCH = 2
CHPX = PPW // NCH


def _sc_body(
    verts_hbm, idx_hbm, bary_hbm, out_hbm,
    idx_v, bary_v, vbuf, obuf0, obuf1, osem0, osem1,
):
    obuf = (obuf0, obuf1)
    osem = (osem0, osem1)
    wid = lax.axis_index("s") * NC + lax.axis_index("c")
    base_px = wid * PPW
    row0 = wid * HPW

    pltpu.sync_copy(idx_hbm.at[:, pl.ds(base_px, PPW)], idx_v)
    pltpu.sync_copy(bary_hbm.at[:, pl.ds(base_px, PPW)], bary_v)

    iota3 = lax.iota(jnp.int32, L) * 3

    def make_px_body(oslot, chunk):
        def px_body(i, _):
            s = chunk * CHPX + i * L
            i0 = idx_v[0, pl.ds(s, L)]
            i1 = idx_v[1, pl.ds(s, L)]
            i2 = idx_v[2, pl.ds(s, L)]
            b0 = bary_v[0, pl.ds(s, L)]
            b1 = bary_v[1, pl.ds(s, L)]
            b2 = bary_v[2, pl.ds(s, L)]
            for c in range(C):
                g0 = plsc.load_gather(vbuf, [i0 + c])
                g1 = plsc.load_gather(vbuf, [i1 + c])
                g2 = plsc.load_gather(vbuf, [i2 + c])
                acc = b0 * g0 + b1 * g1 + b2 * g2
                plsc.store_scatter(
                    obuf[oslot], [iota3 + (i * (L * C) + c)], acc
                )
            return _

        return px_body

    ocopies = [None, None]
    for b in range(B):
        pltpu.sync_copy(verts_hbm.at[b], vbuf)
        for chunk in range(NCH):
            oslot = (b * NCH + chunk) % 2
            if ocopies[oslot] is not None:
                ocopies[oslot].wait()
            lax.fori_loop(
                0, CHPX // L, make_px_body(oslot, chunk), 0, unroll=2
            )
            ocopies[oslot] = pltpu.async_copy(
                obuf[oslot],
                out_hbm.at[b, pl.ds((base_px + chunk * CHPX) * C, CHPX * C)],
                osem[oslot],
            )
    for oc in ocopies:
        if oc is not None:
            oc.wait()


@functools.partial(jax.jit, static_argnames=())
def kernel(verts, bary_weights, v_index):
    idx3 = (v_index.reshape(P, C).astype(jnp.int32) * 3).T
    bary = bary_weights.reshape(P, C).T
    verts_flat = jnp.pad(
        verts.reshape(B, NV * C), ((0, 0), (0, NVP - NV * C))
    )

    sc = pl.kernel(
        _sc_body,
        mesh=plsc.VectorSubcoreMesh(core_axis_name="c", subcore_axis_name="s"),
        out_type=jax.ShapeDtypeStruct((B, P * C), jnp.float32),
        scratch_types=[
            pltpu.VMEM((C, PPW), jnp.int32),
            pltpu.VMEM((C, PPW), jnp.float32),
            pltpu.VMEM((NVP,), jnp.float32),
            pltpu.VMEM((CHPX * C,), jnp.float32),
            pltpu.VMEM((CHPX * C,), jnp.float32),
            pltpu.SemaphoreType.DMA,
            pltpu.SemaphoreType.DMA,
        ],
        compiler_params=pltpu.CompilerParams(needs_layout_passes=False),
    )
    return sc(verts_flat, idx3, bary).reshape(B, H, W, C)


# speedup vs baseline: 3.3237x; 2.8428x over previous
import functools

import jax
import jax.numpy as jnp
from jax import lax
from jax.experimental import pallas as pl
from jax.experimental.pallas import tpu as pltpu
from jax.experimental.pallas import tpu_sc as plsc

B = 16
NV = 6890
H = 512
W = 512
C = 3
P = H * W

_info = plsc.get_sparse_core_info()
NC = _info.num_cores
NS = _info.num_subcores
L = _info.num_lanes
NW = NC * NS
PPW = P // NW
HPW = H // NW
NVP = ((NV * C + 15) // 16) * 16
NCH = 2
CHPX = PPW // NCH


def _sc_body(
    verts_hbm, idx_hbm, bary_hbm, out_hbm,
    idx_v, bary_v, vbuf, obuf0, obuf1, osem0, osem1,
):
    obuf = (obuf0, obuf1)
    osem = (osem0, osem1)
    wid = lax.axis_index("s") * NC + lax.axis_index("c")
    base_px = wid * PPW
    row0 = wid * HPW

    pltpu.sync_copy(idx_hbm.at[:, pl.ds(base_px, PPW)], idx_v)
    pltpu.sync_copy(bary_hbm.at[:, pl.ds(base_px, PPW)], bary_v)

    iota = lax.iota(jnp.int32, L)

    def make_px_body(oslot, chunk):
        def px_body(i, _):
            s = chunk * CHPX + i * L
            i0 = idx_v[0, pl.ds(s, L)]
            i1 = idx_v[1, pl.ds(s, L)]
            i2 = idx_v[2, pl.ds(s, L)]
            b0 = bary_v[0, pl.ds(s, L)]
            b1 = bary_v[1, pl.ds(s, L)]
            b2 = bary_v[2, pl.ds(s, L)]
            lr = i // (W // L)
            lrv = jnp.full((L,), lr, jnp.int32)
            cv = (i % (W // L)) * (L * C) + iota * C
            for c in range(C):
                g0 = plsc.load_gather(vbuf, [i0 + c])
                g1 = plsc.load_gather(vbuf, [i1 + c])
                g2 = plsc.load_gather(vbuf, [i2 + c])
                acc = b0 * g0 + b1 * g1 + b2 * g2
                plsc.store_scatter(obuf[oslot], [lrv, cv + c], acc)
            return _

        return px_body

    ocopies = [None, None]
    for b in range(B):
        pltpu.sync_copy(verts_hbm.at[b], vbuf)
        for chunk in range(NCH):
            oslot = (b * NCH + chunk) % 2
            if ocopies[oslot] is not None:
                ocopies[oslot].wait()
            lax.fori_loop(
                0, CHPX // L, make_px_body(oslot, chunk), 0, unroll=2
            )
            ocopies[oslot] = pltpu.async_copy(
                obuf[oslot],
                out_hbm.at[b, pl.ds(row0 + chunk * (HPW // NCH), HPW // NCH)],
                osem[oslot],
            )
    for oc in ocopies:
        if oc is not None:
            oc.wait()


@functools.partial(jax.jit, static_argnames=())
def kernel(verts, bary_weights, v_index):
    idx3 = (v_index.reshape(P, C).astype(jnp.int32) * 3).T
    bary = bary_weights.reshape(P, C).T
    verts_flat = jnp.pad(
        verts.reshape(B, NV * C), ((0, 0), (0, NVP - NV * C))
    )

    sc = pl.kernel(
        _sc_body,
        mesh=plsc.VectorSubcoreMesh(core_axis_name="c", subcore_axis_name="s"),
        out_type=jax.ShapeDtypeStruct((B, H, W * C), jnp.float32),
        scratch_types=[
            pltpu.VMEM((C, PPW), jnp.int32),
            pltpu.VMEM((C, PPW), jnp.float32),
            pltpu.VMEM((NVP,), jnp.float32),
            pltpu.VMEM((HPW // NCH, W * C), jnp.float32),
            pltpu.VMEM((HPW // NCH, W * C), jnp.float32),
            pltpu.SemaphoreType.DMA,
            pltpu.SemaphoreType.DMA,
        ],
        compiler_params=pltpu.CompilerParams(needs_layout_passes=False),
    )
    return sc(verts_flat, idx3, bary).reshape(B, H, W, C)


# speedup vs baseline: 3.3597x; 1.0109x over previous
import functools

import jax
import jax.numpy as jnp
from jax import lax
from jax.experimental import pallas as pl
from jax.experimental.pallas import tpu as pltpu
from jax.experimental.pallas import tpu_sc as plsc

B = 16
NV = 6890
H = 512
W = 512
C = 3
P = H * W

_info = plsc.get_sparse_core_info()
NC = _info.num_cores
NS = _info.num_subcores
L = _info.num_lanes
NW = NC * NS
PPW = P // NW
HPW = H // NW
NVP = ((NV * C + 15) // 16) * 16
NCH = 2
CHPX = PPW // NCH


def _sc_body(
    verts_hbm, idx_hbm, bary_hbm, out_hbm,
    idx_v, bary_v, vbuf, obuf0, obuf1, osem0, osem1,
):
    obuf = (obuf0, obuf1)
    osem = (osem0, osem1)
    wid = lax.axis_index("s") * NC + lax.axis_index("c")
    base_px = wid * PPW
    row0 = wid * HPW

    pltpu.sync_copy(idx_hbm.at[:, pl.ds(base_px, PPW)], idx_v)
    pltpu.sync_copy(bary_hbm.at[:, pl.ds(base_px, PPW)], bary_v)

    iota = lax.iota(jnp.int32, L)

    def make_px_body(oslot, chunk):
        def px_body(i, _):
            s = chunk * CHPX + i * L
            i0 = idx_v[0, pl.ds(s, L)]
            i1 = idx_v[1, pl.ds(s, L)]
            i2 = idx_v[2, pl.ds(s, L)]
            b0 = bary_v[0, pl.ds(s, L)]
            b1 = bary_v[1, pl.ds(s, L)]
            b2 = bary_v[2, pl.ds(s, L)]
            lr = i // (W // L)
            lrv = jnp.full((L,), lr, jnp.int32)
            cv = (i % (W // L)) * (L * C) + iota * C
            for c in range(C):
                g0 = plsc.load_gather(vbuf, [i0 + c])
                g1 = plsc.load_gather(vbuf, [i1 + c])
                g2 = plsc.load_gather(vbuf, [i2 + c])
                acc = b0 * g0 + b1 * g1 + b2 * g2
                plsc.store_scatter(obuf[oslot], [lrv, cv + c], acc)
            return _

        return px_body

    ocopies = [None, None]
    for b in range(B):
        pltpu.sync_copy(verts_hbm.at[b], vbuf)
        for chunk in range(NCH):
            oslot = (b * NCH + chunk) % 2
            if ocopies[oslot] is not None:
                ocopies[oslot].wait()
            lax.fori_loop(
                0, CHPX // L, make_px_body(oslot, chunk), 0, unroll=4
            )
            ocopies[oslot] = pltpu.async_copy(
                obuf[oslot],
                out_hbm.at[b, pl.ds(row0 + chunk * (HPW // NCH), HPW // NCH)],
                osem[oslot],
            )
    for oc in ocopies:
        if oc is not None:
            oc.wait()


@functools.partial(jax.jit, static_argnames=())
def kernel(verts, bary_weights, v_index):
    idx3 = (v_index.reshape(P, C).astype(jnp.int32) * 3).T
    bary = bary_weights.reshape(P, C).T
    verts_flat = jnp.pad(
        verts.reshape(B, NV * C), ((0, 0), (0, NVP - NV * C))
    )

    sc = pl.kernel(
        _sc_body,
        mesh=plsc.VectorSubcoreMesh(core_axis_name="c", subcore_axis_name="s"),
        out_type=jax.ShapeDtypeStruct((B, H, W * C), jnp.float32),
        scratch_types=[
            pltpu.VMEM((C, PPW), jnp.int32),
            pltpu.VMEM((C, PPW), jnp.float32),
            pltpu.VMEM((NVP,), jnp.float32),
            pltpu.VMEM((HPW // NCH, W * C), jnp.float32),
            pltpu.VMEM((HPW // NCH, W * C), jnp.float32),
            pltpu.SemaphoreType.DMA,
            pltpu.SemaphoreType.DMA,
        ],
        compiler_params=pltpu.CompilerParams(needs_layout_passes=False),
    )
    return sc(verts_flat, idx3, bary).reshape(B, H, W, C)
